# baseline (speedup 1.0000x reference)
import functools

import jax
import jax.numpy as jnp
from jax import lax
from jax.experimental import pallas as pl
from jax.experimental.pallas import tpu as pltpu
from jax.experimental.pallas import tpu_sc as plsc

N_RAYS = 131072
N_BINS = 128
N_SAMPLES = 128
OUT_N = N_BINS + N_SAMPLES
EPS = 1e-5
L = 16
NCHUNK = N_BINS // L

NUM_CORES = 2
NUM_SUBCORES = 16
NWORK = NUM_CORES * NUM_SUBCORES
RAYS_PER_W = N_RAYS // NWORK
CH = 64
NCH = RAYS_PER_W // CH
RAY_ILP = 1
NBUF = 2


def _phase1(row, w_v, ts_v, u_v, u_c8, lane, rampf, cdf_v, hist_v):
    roff = row * N_BINS
    rowsplat = jnp.full((L,), roff, jnp.int32)
    wv = [w_v[pl.ds(roff + L * i, L)] for i in range(NCHUNK)]
    wv[NCHUNK - 1] = jnp.where(lane < L - 1, wv[NCHUNK - 1], 0.0)

    tot = [jnp.sum(wv[i]) for i in range(NCHUNK)]
    ws0 = tot[0]
    for i in range(1, NCHUNK):
        ws0 = ws0 + tot[i]
    pad = jnp.maximum(EPS - ws0, 0.0)
    padj = pad * (1.0 / (N_BINS - 1))
    inv = 1.0 / jnp.full((L,), ws0 + pad, jnp.float32)

    zeros16 = jnp.zeros((L,), jnp.int32)
    for i in range(NCHUNK):
        hist_v[pl.ds(L * i, L)] = zeros16

    carry = jnp.float32(0.0)
    ones = jnp.full((L,), 1, jnp.int32)
    for i in range(NCHUNK):
        cs = plsc.cumsum(wv[i]) + (padj * rampf + (carry + (L * i) * padj))
        if i < NCHUNK - 1:
            carry = carry + tot[i]
        cdfc = jnp.minimum(cs * inv, 1.0)
        plsc.store_scatter(cdf_v, [lane + (L * i + 1)], cdfc)
        z = cdfc * jnp.float32(N_SAMPLES - 1)
        kt = z.astype(jnp.int32)
        kt = kt + (kt.astype(jnp.float32) < z).astype(jnp.int32)
        g0 = plsc.load_gather(u_v, [jnp.maximum(kt - 1, 0)])
        g1 = plsc.load_gather(u_v, [kt])
        k_rank = (kt - 1) + (g0 < cdfc).astype(jnp.int32) + (
            g1 < cdfc
        ).astype(jnp.int32)
        k_rank = jnp.maximum(k_rank, 0)
        if i == NCHUNK - 1:
            plsc.addupdate_scatter(hist_v, [k_rank], ones, mask=lane < L - 1)
        else:
            plsc.addupdate_scatter(hist_v, [k_rank], ones)

    hv = [hist_v[pl.ds(L * i, L)] for i in range(NCHUNK)]
    htot = [jnp.sum(hv[i]) for i in range(NCHUNK)]
    new = []
    icarry = jnp.int32(1)
    for si in range(NCHUNK):
        u_c = u_c8[si]
        lo = plsc.cumsum(hv[si]) + icarry
        if si < NCHUNK - 1:
            icarry = icarry + htot[si]
        below = jnp.maximum(lo - 1, 0)
        above = jnp.minimum(lo, N_BINS - 1)
        c0 = plsc.load_gather(cdf_v, [below])
        c1 = plsc.load_gather(cdf_v, [above])
        b0 = plsc.load_gather(ts_v, [rowsplat + below])
        b1 = plsc.load_gather(ts_v, [rowsplat + above])
        denom = c1 - c0
        denom = jnp.where(denom < EPS, 1.0, denom)
        t = (u_c - c0) / denom
        new.append(b0 + t * (b1 - b0))
    return tuple(new)


def _phase2(row, new, ts_v, out_v):
    roff = row * N_BINS
    ooff = row * OUT_N
    x = [ts_v[pl.ds(roff + L * i, L)] for i in range(NCHUNK)]
    x += [lax.rev(new[NCHUNK - 1 - i], (0,)) for i in range(NCHUNK)]
    nv = len(x)
    stride = nv // 2
    while stride >= 1:
        for base in range(0, nv, 2 * stride):
            for off in range(stride):
                p, q = base + off, base + off + stride
                av, bv = x[p], x[q]
                x[p] = jnp.minimum(av, bv)
                x[q] = jnp.maximum(av, bv)
        stride //= 2
    for i in range(nv):
        out_v[pl.ds(ooff + L * i, L)] = jnp.sort(x[i])


def _sc_kernel(w_hbm, ts_hbm, u_hbm, out_hbm, w_v, ts_v, out_v, u_v, cdf_v,
               hist_v, sem_w, sem_t, sem_o):
    wid = lax.axis_index("s") * NUM_CORES + lax.axis_index("c")
    base0 = wid * RAYS_PER_W
    pltpu.sync_copy(u_hbm, u_v)
    lane = lax.iota(jnp.int32, L)
    rampf = (lane + 1).astype(jnp.float32)
    plsc.store_scatter(cdf_v, [lane], jnp.zeros((L,), jnp.float32))
    for i in range(9):
        hist_v[pl.ds(L * i, L)] = jnp.zeros((L,), jnp.int32)
    u_c8 = [u_v[pl.ds(L * i, L)] for i in range(NCHUNK)]

    def in_copies(c, b):
        base = (base0 + c * CH) * N_BINS
        return (
            pltpu.make_async_copy(
                w_hbm.at[pl.ds(base, CH * N_BINS)],
                w_v.at[pl.ds(b * (CH * N_BINS), CH * N_BINS)], sem_w.at[b]),
            pltpu.make_async_copy(
                ts_hbm.at[pl.ds(base, CH * N_BINS)],
                ts_v.at[pl.ds(b * (CH * N_BINS), CH * N_BINS)], sem_t.at[b]),
        )

    def out_copy(c, b):
        base = (base0 + c * CH) * OUT_N
        return pltpu.make_async_copy(
            out_v.at[pl.ds(b * (CH * OUT_N), CH * OUT_N)],
            out_hbm.at[pl.ds(base, CH * OUT_N)], sem_o.at[b])

    def start_in(c, b):
        for cp in in_copies(c, b):
            cp.start()

    def wait_in(c, b):
        for cp in in_copies(c, b):
            cp.wait()

    for b in range(NBUF):
        start_in(b, b)

    def group_body(c, _):
        b = c % NBUF
        base = base0 + c * CH
        wait_in(c, b)

        @pl.when(c >= NBUF)
        def _drain():
            out_copy(c, b).wait()

        row0 = b * CH

        def ray_body(rr, carry):
            new = _phase1(row0 + rr, w_v, ts_v, u_v, u_c8, lane, rampf,
                          cdf_v, hist_v)
            _phase2(row0 + rr, new, ts_v, out_v)
            return carry

        lax.fori_loop(0, CH, ray_body, 0, unroll=False)
        out_copy(c, b).start()

        @pl.when(c + NBUF < NCH)
        def _prefetch():
            start_in(c + NBUF, b)
        return _

    lax.fori_loop(0, NCH, group_body, 0, unroll=False)
    for b in range(NBUF):
        out_copy(NCH - NBUF + b, b).wait()


@jax.jit
def kernel(weights, ts):
    u = jnp.linspace(0.0, 1.0, N_SAMPLES, dtype=jnp.float32)
    mesh = plsc.VectorSubcoreMesh(
        core_axis_name="c", subcore_axis_name="s"
    )
    f = pl.kernel(
        _sc_kernel,
        out_type=jax.ShapeDtypeStruct((N_RAYS * OUT_N,), jnp.float32),
        mesh=mesh,
        scratch_types=[
            pltpu.VMEM((NBUF * CH * N_BINS,), jnp.float32),
            pltpu.VMEM((NBUF * CH * N_BINS,), jnp.float32),
            pltpu.VMEM((NBUF * CH * OUT_N,), jnp.float32),
            pltpu.VMEM((N_SAMPLES,), jnp.float32),
            pltpu.VMEM((144,), jnp.float32),
            pltpu.VMEM((144,), jnp.int32),
            pltpu.SemaphoreType.DMA((NBUF,)),
            pltpu.SemaphoreType.DMA((NBUF,)),
            pltpu.SemaphoreType.DMA((NBUF,)),
        ],
        compiler_params=pltpu.CompilerParams(needs_layout_passes=False),
    )
    out = f(weights.reshape(-1), ts.reshape(-1), u)
    return out.reshape(N_RAYS, OUT_N)


# speedup vs baseline: 1.1325x; 1.1325x over previous
import jax
import jax.numpy as jnp
from jax import lax
from jax.experimental import pallas as pl
from jax.experimental.pallas import tpu as pltpu
from jax.experimental.pallas import tpu_sc as plsc

N_RAYS = 131072
N_BINS = 128
N_SAMPLES = 128
OUT_N = N_BINS + N_SAMPLES
EPS = 1e-5
L = 16
NCHUNK = N_BINS // L

NUM_CORES = 2
NUM_SUBCORES = 16
NWORK = NUM_CORES * NUM_SUBCORES
RAYS_PER_W = N_RAYS // NWORK
CH = 64
NCH = RAYS_PER_W // CH
NBUF = 2


def _ray_body(r, b, w_v, ts_v, u_v, u_c8, lane, rampf, cdf_v, hist_v, out_v):
    jsplat = jnp.full((L,), 0, jnp.int32)
    bsplat = jnp.full((L,), b, jnp.int32)
    wv = [w_v[b, r, pl.ds(L * i, L)] for i in range(NCHUNK)]
    wv[NCHUNK - 1] = jnp.where(lane < L - 1, wv[NCHUNK - 1], 0.0)

    tot = [jnp.sum(wv[i]) for i in range(NCHUNK)]
    ws0 = tot[0]
    for i in range(1, NCHUNK):
        ws0 = ws0 + tot[i]
    pad = jnp.maximum(EPS - ws0, 0.0)
    padj = pad * (1.0 / (N_BINS - 1))
    inv = 1.0 / jnp.full((L,), ws0 + pad, jnp.float32)

    hist_v[0, pl.ds(0, L)] = jnp.where(lane == 0, 1, 0).astype(jnp.int32)
    for i in range(1, 9):
        hist_v[0, pl.ds(L * i, L)] = jnp.zeros((L,), jnp.int32)

    carry = jnp.float32(0.0)
    ones = jnp.full((L,), 1, jnp.int32)
    for i in range(NCHUNK):
        cs = plsc.cumsum(wv[i]) + (padj * rampf + (carry + (L * i) * padj))
        if i < NCHUNK - 1:
            carry = carry + tot[i]
        cdfc = jnp.minimum(cs * inv, 1.0)
        plsc.store_scatter(cdf_v, [jsplat, lane + (L * i + 1)], cdfc)
        z = cdfc * jnp.float32(N_SAMPLES - 1)
        kt = z.astype(jnp.int32)
        kt = kt + (kt.astype(jnp.float32) < z).astype(jnp.int32)
        g0 = plsc.load_gather(u_v, [jnp.maximum(kt - 1, 0)])
        g1 = plsc.load_gather(u_v, [kt])
        k_rank = (kt - 1) + (g0 < cdfc).astype(jnp.int32) + (
            g1 < cdfc
        ).astype(jnp.int32)
        k_rank = jnp.maximum(k_rank, 0)
        if i == NCHUNK - 1:
            plsc.addupdate_scatter(hist_v, [jsplat, k_rank], ones,
                                   mask=lane < L - 1)
        else:
            plsc.addupdate_scatter(hist_v, [jsplat, k_rank], ones)

    hv = [hist_v[0, pl.ds(L * i, L)] for i in range(NCHUNK)]
    htot = [jnp.sum(hv[i]) for i in range(NCHUNK)]
    new = []
    rsplat = jnp.full((L,), r, jnp.int32)
    icarry = jnp.int32(0)
    for si in range(NCHUNK):
        u_c = u_c8[si]
        lo = plsc.cumsum(hv[si]) + icarry
        if si < NCHUNK - 1:
            icarry = icarry + htot[si]
        below = jnp.maximum(lo - 1, 0)
        above = jnp.minimum(lo, N_BINS - 1)
        c0 = plsc.load_gather(cdf_v, [jsplat, below])
        c1 = plsc.load_gather(cdf_v, [jsplat, above])
        b0 = plsc.load_gather(ts_v, [bsplat, rsplat, below])
        b1 = plsc.load_gather(ts_v, [bsplat, rsplat, above])
        denom = c1 - c0
        denom = jnp.where(denom < EPS, 1.0, denom)
        t = (u_c - c0) / denom
        new.append(b0 + t * (b1 - b0))

    x = [ts_v[b, r, pl.ds(L * i, L)] for i in range(NCHUNK)]
    x += [lax.rev(new[NCHUNK - 1 - i], (0,)) for i in range(NCHUNK)]
    nv = len(x)
    stride = nv // 2
    while stride >= 1:
        for base in range(0, nv, 2 * stride):
            for off in range(stride):
                p, q = base + off, base + off + stride
                av, bv = x[p], x[q]
                x[p] = jnp.minimum(av, bv)
                x[q] = jnp.maximum(av, bv)
        stride //= 2
    for i in range(nv):
        out_v[b, r, pl.ds(L * i, L)] = jnp.sort(x[i])


def _sc_kernel(w_hbm, ts_hbm, u_hbm, out_hbm, w_v, ts_v, out_v, u_v, cdf_v,
               hist_v, sem_w, sem_t, sem_o):
    wid = lax.axis_index("s") * NUM_CORES + lax.axis_index("c")
    base0 = wid * RAYS_PER_W
    pltpu.sync_copy(u_hbm, u_v)
    lane = lax.iota(jnp.int32, L)
    rampf = (lane + 1).astype(jnp.float32)
    plsc.store_scatter(cdf_v, [jnp.full((L,), 0, jnp.int32), lane],
                       jnp.zeros((L,), jnp.float32))
    u_c8 = [u_v[pl.ds(L * i, L)] for i in range(NCHUNK)]

    def start_in(c, b):
        base = base0 + c * CH
        pltpu.make_async_copy(
            w_hbm.at[pl.ds(base, CH)], w_v.at[b], sem_w.at[b]).start()
        pltpu.make_async_copy(
            ts_hbm.at[pl.ds(base, CH)], ts_v.at[b], sem_t.at[b]).start()

    def wait_in(c, b):
        base = base0 + c * CH
        pltpu.make_async_copy(
            w_hbm.at[pl.ds(base, CH)], w_v.at[b], sem_w.at[b]).wait()
        pltpu.make_async_copy(
            ts_hbm.at[pl.ds(base, CH)], ts_v.at[b], sem_t.at[b]).wait()

    for b in range(NBUF):
        start_in(b, b)

    def group_body(g, _):
        for b in range(NBUF):
            c = g * NBUF + b
            base = base0 + c * CH
            wait_in(c, b)

            @pl.when(g > 0)
            def _drain():
                pltpu.make_async_copy(
                    out_v.at[b], out_hbm.at[pl.ds(base, CH)],
                    sem_o.at[b]).wait()

            def ray_body(rr, carry):
                _ray_body(rr, b, w_v, ts_v, u_v, u_c8, lane, rampf,
                          cdf_v, hist_v, out_v)
                return carry

            lax.fori_loop(0, CH, ray_body, 0, unroll=False)
            pltpu.make_async_copy(
                out_v.at[b], out_hbm.at[pl.ds(base, CH)], sem_o.at[b]).start()

            @pl.when(g < NCH // NBUF - 1)
            def _prefetch():
                start_in(c + NBUF, b)
        return _

    lax.fori_loop(0, NCH // NBUF, group_body, 0, unroll=False)
    for b in range(NBUF):
        pltpu.make_async_copy(
            out_v.at[b], out_hbm.at[pl.ds(base0, CH)], sem_o.at[b]).wait()


@jax.jit
def kernel(weights, ts):
    u = jnp.linspace(0.0, 1.0, N_SAMPLES, dtype=jnp.float32)
    mesh = plsc.VectorSubcoreMesh(
        core_axis_name="c", subcore_axis_name="s"
    )
    f = pl.kernel(
        _sc_kernel,
        out_type=jax.ShapeDtypeStruct((N_RAYS, OUT_N), jnp.float32),
        mesh=mesh,
        scratch_types=[
            pltpu.VMEM((NBUF, CH, N_BINS), jnp.float32),
            pltpu.VMEM((NBUF, CH, N_BINS), jnp.float32),
            pltpu.VMEM((NBUF, CH, OUT_N), jnp.float32),
            pltpu.VMEM((N_SAMPLES,), jnp.float32),
            pltpu.VMEM((1, 144), jnp.float32),
            pltpu.VMEM((1, 144), jnp.int32),
            pltpu.SemaphoreType.DMA((NBUF,)),
            pltpu.SemaphoreType.DMA((NBUF,)),
            pltpu.SemaphoreType.DMA((NBUF,)),
        ],
        compiler_params=pltpu.CompilerParams(needs_layout_passes=False),
    )
    return f(weights, ts, u)
